# baseline (speedup 1.0000x reference)
import jax
import jax.numpy as jnp
from jax import lax
from jax.experimental import pallas as pl
from jax.experimental.pallas import tpu as pltpu
from jax.experimental.pallas import tpu_sc as plsc

D = 64
W2 = 128
L = 16
R = D // L
B = 4096
H = 50
H_PAD = 56
T = 100
T_PAD = 104
G = 0.5
NGRP = 7

_INFO = plsc.get_sparse_core_info()
NC, NS = _INFO.num_cores, _INFO.num_subcores
NW = NC * NS
BPW = B // NW
NPAIR = BPW // 2


def _rsqrt(x):
    i = lax.bitcast_convert_type(x, jnp.int32)
    y = lax.bitcast_convert_type(jnp.int32(0x5F3759DF) - (i >> 1), jnp.float32)
    for _ in range(3):
        y = y * (1.5 - 0.5 * x * y * y)
    return y


def _lanesum(v, lanes):
    for s in (8, 4, 2, 1):
        v = v + v.at[lanes ^ s].get(mode="promise_in_bounds")
    return v


def _body(ugp_hbm, uc_hbm, ig_hbm, ic_hbm, tg_hbm, tc_hbm,
          ue2_hbm, ie2_hbm, wt_hbm, out_hbm,
          ig_f, ic_f, tg_f, tc_f, ugp_f, uc_v, wt_v,
          urp_a, urp_b, hist_a, hist_b, tgt_a, tgt_b,
          ssq_tr, dot_tr, out_st,
          sem_ua, sem_ub, sem_ah, sem_bh, sem_ta, sem_tb):
    wid = lax.axis_index("s") * NC + lax.axis_index("c")
    base = wid * BPW
    pltpu.sync_copy(ig_hbm.at[pl.ds(base * H_PAD, BPW * H_PAD)], ig_f)
    pltpu.sync_copy(ic_hbm.at[pl.ds(base * H_PAD, BPW * H_PAD)], ic_f)
    pltpu.sync_copy(tg_hbm.at[pl.ds(base * T_PAD, BPW * T_PAD)], tg_f)
    pltpu.sync_copy(tc_hbm.at[pl.ds(base * T_PAD, BPW * T_PAD)], tc_f)
    pltpu.sync_copy(ugp_hbm.at[pl.ds(base * 4, NPAIR * 8)], ugp_f)
    pltpu.sync_copy(uc_hbm.at[pl.ds(base, BPW)], uc_v)
    pltpu.sync_copy(wt_hbm, wt_v)

    lanes = lax.iota(jnp.int32, L)
    lanes16 = lanes * L
    bidx = [jnp.full((L,), l, jnp.int32) for l in range(L)]

    def pair_copies(pair, hist_ref, urp_ref, sem_h, sem_u):
        ib = pl.multiple_of(pair * (2 * H_PAD), 8)
        ub = pl.multiple_of(pair * 8, 8)
        return [
            pltpu.make_async_copy(ie2_hbm.at[ig_f.at[pl.ds(ib, H)]],
                                  hist_ref.at[pl.ds(0, H)], sem_h),
            pltpu.make_async_copy(ie2_hbm.at[ig_f.at[pl.ds(ib + H_PAD, H)]],
                                  hist_ref.at[pl.ds(H_PAD, H)], sem_h),
            pltpu.make_async_copy(ue2_hbm.at[ugp_f.at[pl.ds(ub, 8)]],
                                  urp_ref, sem_u),
        ]

    def issue_pair(pair, hist_ref, urp_ref, sem_h, sem_u):
        for cp in pair_copies(pair, hist_ref, urp_ref, sem_h, sem_u):
            cp.start()

    def drain_pair(pair, hist_ref, urp_ref, sem_h, sem_u):
        for cp in pair_copies(pair, hist_ref, urp_ref, sem_h, sem_u):
            cp.wait()

    def tgt_copy(u, tref, sem_t):
        tb = pl.multiple_of(u * T_PAD, 8)
        return pltpu.make_async_copy(ie2_hbm.at[tg_f.at[pl.ds(tb, T_PAD)]],
                                     tref, sem_t)

    def pooled(b, hist_ref, h0):
        ib = pl.multiple_of(b * H_PAD, 8)

        def nz(off, n=None):
            g = ig_f[pl.ds(ib + off, L)]
            c = ic_f[pl.ds(ib + off, L)]
            return (g != 0) | (c != 0)

        cacc = jnp.where(nz(0), 1.0, 0.0)
        cacc = cacc + jnp.where(nz(L), 1.0, 0.0)
        cacc = cacc + jnp.where(nz(2 * L), 1.0, 0.0)
        cacc = cacc + jnp.where((lanes >= 8) & (lanes <= 9) & nz(40),
                                1.0, 0.0)
        inv = 1.0 / _lanesum(cacc, lanes)

        def chunk(c, accs):
            r0 = 10 * c
            out = list(accs)
            for i in range(10):
                cv = plsc.load_gather(ic_f, [lanes * 0 + (ib + r0 + i)])
                co = pl.multiple_of(cv[0], 64)
                for r in range(R):
                    out[r] = out[r] + hist_ref[h0 + r0 + i,
                                               pl.ds(co + L * r, L)]
            return tuple(out)
        accs = lax.fori_loop(
            0, H // 10, chunk,
            tuple(jnp.zeros((L,), jnp.float32) for _ in range(R)))
        return [a * inv for a in accs]

    def finish(b, tref, sem_t, urp_ref, urow, hs):
        tgt_copy(b, tref, sem_t).wait()
        ucv = plsc.load_gather(uc_v, [lanes * 0 + b])
        uco = pl.multiple_of(ucv[0], 64)
        uv = [G * urp_ref[urow, pl.ds(uco + L * r, L)] + (1.0 - G) * hs[r]
              for r in range(R)]
        un = uv[0] * uv[0]
        for r in range(1, R):
            un = un + uv[r] * uv[r]
        ussq = _lanesum(un, lanes)

        tb = pl.multiple_of(b * T_PAD, 8)

        def grp(gi, carry):
            n0 = pl.multiple_of(jnp.minimum(L * gi, 88), 8)
            for j in range(L):
                cv = plsc.load_gather(tc_f, [lanes * 0 + (tb + n0 + j)])
                co = pl.multiple_of(cv[0], 64)
                t = [tref[n0 + j, pl.ds(co + L * r, L)] for r in range(R)]
                sv = t[0] * t[0]
                dv = t[0] * uv[0]
                for r in range(1, R):
                    sv = sv + t[r] * t[r]
                    dv = dv + t[r] * uv[r]
                plsc.store_scatter(ssq_tr, [lanes16 + j], sv)
                plsc.store_scatter(dot_tr, [lanes16 + j], dv)
            ssqv = ssq_tr[pl.ds(0, L)]
            dotv = dot_tr[pl.ds(0, L)]
            for l in range(1, L):
                ssqv = ssqv + ssq_tr[pl.ds(L * l, L)]
                dotv = dotv + dot_tr[pl.ds(L * l, L)]
            x = jnp.maximum(ssqv * ussq, 1e-30)
            out_st[urow, pl.ds(n0, L)] = dotv * _rsqrt(x)
            return carry
        lax.fori_loop(0, NGRP, grp, 0)

        @pl.when(b + 2 < BPW)
        def _():
            tgt_copy(b + 2, tref, sem_t).start()

    def compute_pair(pair, hist_ref, urp_ref, sem_h, sem_u):
        b0 = 2 * pair
        drain_pair(pair, hist_ref, urp_ref, sem_h, sem_u)
        pr_a = pooled(b0, hist_ref, 0)
        pr_b = pooled(b0 + 1, hist_ref, H_PAD)
        hs_a = [jnp.zeros((L,), jnp.float32) for _ in range(R)]
        hs_b = [jnp.zeros((L,), jnp.float32) for _ in range(R)]
        for q in range(R):
            for l in range(L):
                k = L * q + l
                pa = pr_a[q].at[bidx[l]].get(mode="promise_in_bounds")
                pb = pr_b[q].at[bidx[l]].get(mode="promise_in_bounds")
                for r in range(R):
                    w = wt_v[k, pl.ds(L * r, L)]
                    hs_a[r] = hs_a[r] + pa * w
                    hs_b[r] = hs_b[r] + pb * w
        finish(b0, tgt_a, sem_ta, urp_ref, 0, hs_a)
        finish(b0 + 1, tgt_b, sem_tb, urp_ref, 1, hs_b)
        pltpu.sync_copy(out_st, out_hbm.at[pl.ds(b0 + base, 2)])

    issue_pair(0, hist_a, urp_a, sem_ah, sem_ua)
    tgt_copy(0, tgt_a, sem_ta).start()
    tgt_copy(1, tgt_b, sem_tb).start()

    def outer(g, carry):
        pair_a = 2 * g
        pair_b = 2 * g + 1
        issue_pair(pair_b, hist_b, urp_b, sem_bh, sem_ub)
        compute_pair(pair_a, hist_a, urp_a, sem_ah, sem_ua)

        @pl.when(g < NPAIR // 2 - 1)
        def _():
            issue_pair(pair_a + 2, hist_a, urp_a, sem_ah, sem_ua)
        compute_pair(pair_b, hist_b, urp_b, sem_bh, sem_ub)
        return carry

    lax.fori_loop(0, NPAIR // 2, outer, 0)


def kernel(user_idx, interacted_items, target_idx, user_emb, item_emb, W):
    i32 = jnp.int32
    ui = user_idx.astype(i32)
    ug = ui >> 1
    uc = (ui & 1) * 64
    pspread = ((jnp.arange(B // 2, dtype=i32) * 509) % 999983 + 1) >> 1
    ugp = jnp.concatenate(
        [ug.reshape(B // 2, 2),
         jnp.broadcast_to(pspread[:, None], (B // 2, 6))], axis=1).reshape(-1)
    iip = jnp.pad(interacted_items.astype(i32), ((0, 0), (0, H_PAD - H)))
    ig = (iip >> 1).reshape(-1)
    ic = ((iip & 1) * 64).reshape(-1)
    spread = (jnp.arange(B, dtype=i32) * 509) % 999983 + 1
    ti2 = jnp.concatenate(
        [target_idx.astype(i32),
         jnp.broadcast_to(spread[:, None], (B, T_PAD - T))], axis=1)
    tg = (ti2 >> 1).reshape(-1)
    tc = ((ti2 & 1) * 64).reshape(-1)
    ue2 = user_emb.astype(jnp.float32).reshape(500000, W2)
    ie2 = item_emb.astype(jnp.float32).reshape(500000, W2)
    wt = W.T.astype(jnp.float32)
    mesh = plsc.VectorSubcoreMesh(core_axis_name="c", subcore_axis_name="s")
    run = pl.kernel(
        _body,
        out_type=jax.ShapeDtypeStruct((B, W2), jnp.float32),
        mesh=mesh,
        compiler_params=pltpu.CompilerParams(needs_layout_passes=False,
                                             use_tc_tiling_on_sc=True),
        scratch_types=[
            pltpu.VMEM((BPW * H_PAD,), jnp.int32),
            pltpu.VMEM((BPW * H_PAD,), jnp.int32),
            pltpu.VMEM((BPW * T_PAD,), jnp.int32),
            pltpu.VMEM((BPW * T_PAD,), jnp.int32),
            pltpu.VMEM((NPAIR * 8,), jnp.int32),
            pltpu.VMEM((BPW,), jnp.int32),
            pltpu.VMEM((D, D), jnp.float32),
            pltpu.VMEM((8, W2), jnp.float32),
            pltpu.VMEM((8, W2), jnp.float32),
            pltpu.VMEM((2 * H_PAD, W2), jnp.float32),
            pltpu.VMEM((2 * H_PAD, W2), jnp.float32),
            pltpu.VMEM((T_PAD, W2), jnp.float32),
            pltpu.VMEM((T_PAD, W2), jnp.float32),
            pltpu.VMEM((L * L,), jnp.float32),
            pltpu.VMEM((L * L,), jnp.float32),
            pltpu.VMEM((2, W2), jnp.float32),
            pltpu.SemaphoreType.DMA,
            pltpu.SemaphoreType.DMA,
            pltpu.SemaphoreType.DMA,
            pltpu.SemaphoreType.DMA,
            pltpu.SemaphoreType.DMA,
            pltpu.SemaphoreType.DMA,
        ],
    )
    out = run(ugp, uc, ig, ic, tg, tc, ue2, ie2, wt)
    return out[:, :T]


# speedup vs baseline: 1.1160x; 1.1160x over previous
import jax
import jax.numpy as jnp
from jax import lax
from jax.experimental import pallas as pl
from jax.experimental.pallas import tpu as pltpu
from jax.experimental.pallas import tpu_sc as plsc

D = 64
L = 16
R = D // L
B = 4096
H = 50
H_PAD = 56
T = 100
T_PAD = 104
G = 0.5
NGRP = 7

_INFO = plsc.get_sparse_core_info()
NC, NS = _INFO.num_cores, _INFO.num_subcores
NW = NC * NS
BPW = B // NW
NPAIR = BPW // 2


def _rsqrt(x):
    i = lax.bitcast_convert_type(x, jnp.int32)
    y = lax.bitcast_convert_type(jnp.int32(0x5F3759DF) - (i >> 1), jnp.float32)
    for _ in range(3):
        y = y * (1.5 - 0.5 * x * y * y)
    return y


def _lanesum(v, lanes):
    for s in (8, 4, 2, 1):
        v = v + v.at[lanes ^ s].get(mode="promise_in_bounds")
    return v


_HSPLIT = ((0, 32), (32, 32), (64, 24), (88, 24))
_TSPLIT = ((0, 32), (32, 32), (64, 24), (88, 16))


def _hist_copies(pair, hist_ref, sem_h, ii_f, ie_hbm):
    ihb = pl.multiple_of(pair * (2 * H_PAD), 8)
    return [pltpu.make_async_copy(
        ie_hbm.at[ii_f.at[pl.ds(ihb + o, c)]],
        hist_ref.at[pl.ds(o, c)], sem_h) for o, c in _HSPLIT]


def _tgt_copies(pair, tgt_ref, sem_t, ti_f, ie_hbm):
    itb = pl.multiple_of(pair * (2 * T_PAD), 8)
    return [pltpu.make_async_copy(
        ie_hbm.at[ti_f.at[pl.ds(itb + u * T_PAD + o, c)]],
        tgt_ref.at[pl.ds(u * T_PAD + o, c)], sem_t)
        for u in (0, 1) for o, c in _TSPLIT]


def _issue(pair, hist_ref, tgt_ref, sem_h, sem_t, ii_f, ti_f, ie_hbm):
    for cp in _hist_copies(pair, hist_ref, sem_h, ii_f, ie_hbm):
        cp.start()
    for cp in _tgt_copies(pair, tgt_ref, sem_t, ti_f, ie_hbm):
        cp.start()


def _drain_hist(pair, hist_ref, sem_h, ii_f, ie_hbm):
    for cp in _hist_copies(pair, hist_ref, sem_h, ii_f, ie_hbm):
        cp.wait()


def _drain_tgt(pair, tgt_ref, sem_t, ti_f, ie_hbm):
    for cp in _tgt_copies(pair, tgt_ref, sem_t, ti_f, ie_hbm):
        cp.wait()


def _body(ui_hbm, ii_hbm, ti_hbm, ue_hbm, ie_hbm, wt_hbm, out_hbm,
          ii_f, ti_f, ui_v, wt_v, urows_v, hist_a, hist_b, tgt_a, tgt_b,
          ssq_tr, dot_tr, out_v, sem_u, sem_ah, sem_at, sem_bh, sem_bt):
    wid = lax.axis_index("s") * NC + lax.axis_index("c")
    base = wid * BPW
    pltpu.sync_copy(ii_hbm.at[pl.ds(base * H_PAD, BPW * H_PAD)], ii_f)
    pltpu.sync_copy(ti_hbm.at[pl.ds(base * T_PAD, BPW * T_PAD)], ti_f)
    pltpu.sync_copy(ui_hbm.at[pl.ds(base, BPW)], ui_v)
    pltpu.sync_copy(wt_hbm, wt_v)
    ucopy = pltpu.make_async_copy(ue_hbm.at[ui_v], urows_v, sem_u)
    ucopy.start()

    lanes = lax.iota(jnp.int32, L)
    lanes16 = lanes * L
    bidx = [jnp.full((L,), l, jnp.int32) for l in range(L)]

    def pooled(b, hist_ref, h0):
        ib = pl.multiple_of(b * H_PAD, 8)
        cacc = jnp.where(ii_f[pl.ds(ib, L)] != 0, 1.0, 0.0)
        cacc = cacc + jnp.where(ii_f[pl.ds(ib + L, L)] != 0, 1.0, 0.0)
        cacc = cacc + jnp.where(ii_f[pl.ds(ib + 2 * L, L)] != 0, 1.0, 0.0)
        tail = ii_f[pl.ds(ib + 40, L)]
        cacc = cacc + jnp.where((lanes >= 8) & (lanes <= 9) & (tail != 0),
                                1.0, 0.0)
        inv = 1.0 / _lanesum(cacc, lanes)

        def chunk(c, accs):
            r0 = 10 * c
            out = list(accs)
            for i in range(10):
                for r in range(R):
                    out[r] = out[r] + hist_ref[h0 + r0 + i, pl.ds(L * r, L)]
            return tuple(out)
        accs = lax.fori_loop(
            0, H // 10, chunk,
            tuple(jnp.zeros((L,), jnp.float32) for _ in range(R)))
        return [a * inv for a in accs]

    def finish(b, tgt_ref, t0, hs):
        uv = [G * urows_v[b, pl.ds(L * r, L)] + (1.0 - G) * hs[r]
              for r in range(R)]
        un = uv[0] * uv[0]
        for r in range(1, R):
            un = un + uv[r] * uv[r]
        ussq = _lanesum(un, lanes)

        def grp(gi, carry):
            n0 = pl.multiple_of(jnp.minimum(L * gi, 88), 8)
            for j in range(L):
                n = t0 + n0 + j
                t = [tgt_ref[n, pl.ds(L * r, L)] for r in range(R)]
                sv = t[0] * t[0]
                dv = t[0] * uv[0]
                for r in range(1, R):
                    sv = sv + t[r] * t[r]
                    dv = dv + t[r] * uv[r]
                plsc.store_scatter(ssq_tr, [lanes16 + j], sv)
                plsc.store_scatter(dot_tr, [lanes16 + j], dv)
            ssqv = ssq_tr[pl.ds(0, L)]
            dotv = dot_tr[pl.ds(0, L)]
            for l in range(1, L):
                ssqv = ssqv + ssq_tr[pl.ds(L * l, L)]
                dotv = dotv + dot_tr[pl.ds(L * l, L)]
            x = jnp.maximum(ssqv * ussq, 1e-30)
            out_v[b, pl.ds(n0, L)] = dotv * _rsqrt(x)
            return carry
        lax.fori_loop(0, NGRP, grp, 0)

    def compute_pair(pair, hist_ref, tgt_ref, sem_h, sem_t):
        b0 = 2 * pair
        _drain_hist(pair, hist_ref, sem_h, ii_f, ie_hbm)
        pr_a = pooled(b0, hist_ref, 0)
        pr_b = pooled(b0 + 1, hist_ref, H_PAD)
        hs_a = [jnp.zeros((L,), jnp.float32) for _ in range(R)]
        hs_b = [jnp.zeros((L,), jnp.float32) for _ in range(R)]
        for q in range(R):
            for l in range(L):
                k = L * q + l
                pa = pr_a[q].at[bidx[l]].get(mode="promise_in_bounds")
                pb = pr_b[q].at[bidx[l]].get(mode="promise_in_bounds")
                for r in range(R):
                    w = wt_v[k, pl.ds(L * r, L)]
                    hs_a[r] = hs_a[r] + pa * w
                    hs_b[r] = hs_b[r] + pb * w
        _drain_tgt(pair, tgt_ref, sem_t, ti_f, ie_hbm)
        finish(b0, tgt_ref, 0, hs_a)
        finish(b0 + 1, tgt_ref, T_PAD, hs_b)

    _issue(0, hist_a, tgt_a, sem_ah, sem_at, ii_f, ti_f, ie_hbm)
    ucopy.wait()

    def outer(g, carry):
        pair_a = 2 * g
        pair_b = 2 * g + 1
        _issue(pair_b, hist_b, tgt_b, sem_bh, sem_bt, ii_f, ti_f, ie_hbm)
        compute_pair(pair_a, hist_a, tgt_a, sem_ah, sem_at)

        @pl.when(g < NPAIR // 2 - 1)
        def _():
            _issue(pair_a + 2, hist_a, tgt_a, sem_ah, sem_at, ii_f, ti_f,
                   ie_hbm)
        compute_pair(pair_b, hist_b, tgt_b, sem_bh, sem_bt)
        return carry

    lax.fori_loop(0, NPAIR // 2, outer, 0)
    pltpu.sync_copy(out_v, out_hbm.at[pl.ds(base, BPW)])


def kernel(user_idx, interacted_items, target_idx, user_emb, item_emb, W):
    ui = user_idx.astype(jnp.int32)
    spread = (jnp.arange(B, dtype=jnp.int32) * 509) % 999983 + 1
    ii = jnp.concatenate(
        [interacted_items.astype(jnp.int32),
         jnp.broadcast_to(spread[:, None], (B, H_PAD - H))], axis=1
    ).reshape(-1)
    ti = jnp.concatenate(
        [target_idx.astype(jnp.int32),
         jnp.broadcast_to(spread[:, None], (B, T_PAD - T))], axis=1
    ).reshape(-1)
    wt = W.T.astype(jnp.float32)
    mesh = plsc.VectorSubcoreMesh(core_axis_name="c", subcore_axis_name="s")
    run = pl.kernel(
        _body,
        out_type=jax.ShapeDtypeStruct((B, T_PAD), jnp.float32),
        mesh=mesh,
        compiler_params=pltpu.CompilerParams(needs_layout_passes=False,
                                             use_tc_tiling_on_sc=False),
        scratch_types=[
            pltpu.VMEM((BPW * H_PAD,), jnp.int32),
            pltpu.VMEM((BPW * T_PAD,), jnp.int32),
            pltpu.VMEM((BPW,), jnp.int32),
            pltpu.VMEM((D, D), jnp.float32),
            pltpu.VMEM((BPW, D), jnp.float32),
            pltpu.VMEM((2 * H_PAD, D), jnp.float32),
            pltpu.VMEM((2 * H_PAD, D), jnp.float32),
            pltpu.VMEM((2 * T_PAD, D), jnp.float32),
            pltpu.VMEM((2 * T_PAD, D), jnp.float32),
            pltpu.VMEM((L * L,), jnp.float32),
            pltpu.VMEM((L * L,), jnp.float32),
            pltpu.VMEM((BPW, T_PAD), jnp.float32),
            pltpu.SemaphoreType.DMA,
            pltpu.SemaphoreType.DMA,
            pltpu.SemaphoreType.DMA,
            pltpu.SemaphoreType.DMA,
            pltpu.SemaphoreType.DMA,
        ],
    )
    out = run(ui, ii, ti, user_emb.astype(jnp.float32),
              item_emb.astype(jnp.float32), wt)
    return out[:, :T]
